# baseline (speedup 1.0000x reference)
import functools

import jax
import jax.numpy as jnp
from jax.experimental import pallas as pl

_NUM_POINTS = 8192
_INT_MIN = -2147483648


def _uncertainty(x):
    m1 = jnp.max(x, axis=0, keepdims=True)
    eq = x == m1
    cnt = jnp.sum(eq.astype(jnp.int32), axis=0, keepdims=True)
    m2 = jnp.max(jnp.where(eq, -jnp.inf, x), axis=0, keepdims=True)
    second = jnp.where(cnt > 1, m1, m2)
    return second - m1


def _sort_key(u):
    b = jax.lax.bitcast_convert_type(u, jnp.int32)
    return jnp.where(b >= 0, b, ~(b & jnp.int32(0x7FFFFFFF)))


def _find_threshold(k, p):
    n_nonneg = jnp.sum((k >= 0).astype(jnp.int32))
    t0 = jnp.where(n_nonneg >= p, jnp.int32(0), jnp.int32(_INT_MIN))

    def body(i, t):
        cand = t | (jnp.int32(1) << (jnp.int32(30) - i))
        cnt = jnp.sum((k >= cand).astype(jnp.int32))
        return jnp.where(cnt >= p, cand, t)

    return jax.lax.fori_loop(0, 31, body, t0)


def _pool_kernel(p, x0ref, x1ref, plref, prref, cref, oref, tref):
    pl_m = plref[...]
    pr_m = prref[...]
    hw = oref.shape[2]
    cb = x0ref.shape[1]
    for s, xref in enumerate((x0ref, x1ref)):
        for i in range(cb):
            x = xref[0, i]
            t = jnp.dot(pl_m, x, preferred_element_type=jnp.float32)
            y = jnp.dot(t, pr_m, preferred_element_type=jnp.float32) * 0.25
            oref[0, s * cb + i] = y.astype(jnp.bfloat16).reshape(hw)

    @pl.when(jnp.logical_and(pl.program_id(0) == 0, pl.program_id(1) == 0))
    def _():
        for n in range(cref.shape[0]):
            k = _sort_key(_uncertainty(cref[n]))
            tref[n] = jnp.full((1, tref.shape[2]),
                               _find_threshold(k, p), jnp.int32)


def _mlp_kernel(cref, faref, fbref, tref, w1faref, w1fbref, w1cref,
                w2ref, w3ref, b1ref, b2ref, b3ref, oref):
    c = cref[0]
    fa = faref[0, 0]
    fb = fbref[0, 0]
    k = _sort_key(_uncertainty(c))
    t = tref[0][:, 0:1]
    maskf = (k >= t).astype(jnp.float32)
    h1 = jnp.maximum(
        jnp.dot(w1faref[...], fa, preferred_element_type=jnp.float32)
        + jnp.dot(w1fbref[...], fb, preferred_element_type=jnp.float32)
        + jnp.dot(w1cref[...], c, preferred_element_type=jnp.float32)
        + b1ref[...], 0.0)
    h2 = jnp.maximum(
        jnp.dot(w2ref[...], h1.astype(jnp.bfloat16),
                preferred_element_type=jnp.float32)
        + b2ref[...], 0.0)
    lg = jnp.dot(w3ref[...], h2, preferred_element_type=jnp.float32) + b3ref[...]
    oref[0] = lg * maskf + c * (1.0 - maskf)


def kernel(coarse_logits, fine_features, w1, b1, w2, b2, w3, b3):
    N, C, H, W = coarse_logits.shape
    _, Cf, Hf, Wf = fine_features.shape
    HW = H * W
    P = min(HW, _NUM_POINTS)
    hidden = w1.shape[0]
    coarse3 = coarse_logits.reshape(N, C, HW)

    pl_mat = jnp.repeat(jnp.eye(Hf // 2, dtype=jnp.float32), 2, axis=1)
    pr_mat = jnp.repeat(jnp.eye(Wf // 2, dtype=jnp.float32), 2, axis=0)
    cb = 16
    step_c = 2 * cb
    fine_spec = lambda i: pl.BlockSpec(
        (1, cb, Hf, Wf), lambda n, j, i=i: (n, 2 * j + i, 0, 0))
    pooled, thr = pl.pallas_call(
        functools.partial(_pool_kernel, P),
        grid=(N, Cf // step_c),
        in_specs=[fine_spec(0), fine_spec(1),
                  pl.BlockSpec((Hf // 2, Hf), lambda n, j: (0, 0)),
                  pl.BlockSpec((Wf, Wf // 2), lambda n, j: (0, 0)),
                  pl.BlockSpec((N, C, HW), lambda n, j: (0, 0, 0))],
        out_specs=[pl.BlockSpec((1, step_c, HW), lambda n, j: (n, j, 0)),
                   pl.BlockSpec((N, 1, 128), lambda n, j: (0, 0, 0))],
        out_shape=[jax.ShapeDtypeStruct((N, Cf, HW), jnp.bfloat16),
                   jax.ShapeDtypeStruct((N, 1, 128), jnp.int32)],
    )(fine_features, fine_features, pl_mat, pr_mat, coarse3)
    pooled4 = pooled.reshape(N, 2, Cf // 2, HW)

    B = 4096
    nb = HW // B
    Cfh = Cf // 2
    w1fa = w1[:, :Cfh].astype(jnp.bfloat16)
    w1fb = w1[:, Cfh:Cf].astype(jnp.bfloat16)
    w1c = w1[:, Cf:]
    w2b = w2.astype(jnp.bfloat16)
    refined = pl.pallas_call(
        _mlp_kernel,
        grid=(N, nb),
        in_specs=[
            pl.BlockSpec((1, C, B), lambda n, b: (n, 0, b)),
            pl.BlockSpec((1, 1, Cfh, B), lambda n, b: (n, 0, 0, b)),
            pl.BlockSpec((1, 1, Cfh, B), lambda n, b: (n, 1, 0, b)),
            pl.BlockSpec((1, 1, 128), lambda n, b: (n, 0, 0)),
            pl.BlockSpec((hidden, Cfh), lambda n, b: (0, 0)),
            pl.BlockSpec((hidden, Cfh), lambda n, b: (0, 0)),
            pl.BlockSpec((hidden, C), lambda n, b: (0, 0)),
            pl.BlockSpec((hidden, hidden), lambda n, b: (0, 0)),
            pl.BlockSpec((C, hidden), lambda n, b: (0, 0)),
            pl.BlockSpec((hidden, 1), lambda n, b: (0, 0)),
            pl.BlockSpec((hidden, 1), lambda n, b: (0, 0)),
            pl.BlockSpec((C, 1), lambda n, b: (0, 0)),
        ],
        out_specs=pl.BlockSpec((1, C, B), lambda n, b: (n, 0, b)),
        out_shape=jax.ShapeDtypeStruct((N, C, HW), jnp.float32),
    )(coarse3, pooled4, pooled4, thr, w1fa, w1fb, w1c, w2b, w3,
      b1.reshape(hidden, 1), b2.reshape(hidden, 1), b3.reshape(C, 1))
    return refined.reshape(N, C, H, W)


# speedup vs baseline: 1.0840x; 1.0840x over previous
import functools

import jax
import jax.numpy as jnp
from jax.experimental import pallas as pl
from jax.experimental.pallas import tpu as pltpu

_NUM_POINTS = 8192
_INT_MIN = -2147483648


def _uncertainty(x):
    m1 = jnp.max(x, axis=0, keepdims=True)
    eq = x == m1
    cnt = jnp.sum(eq.astype(jnp.int32), axis=0, keepdims=True)
    m2 = jnp.max(jnp.where(eq, -jnp.inf, x), axis=0, keepdims=True)
    second = jnp.where(cnt > 1, m1, m2)
    return second - m1


def _sort_key(u):
    b = jax.lax.bitcast_convert_type(u, jnp.int32)
    return jnp.where(b >= 0, b, ~(b & jnp.int32(0x7FFFFFFF)))


def _find_threshold(k, p):
    n_nonneg = jnp.sum((k >= 0).astype(jnp.int32))
    t0 = jnp.where(n_nonneg >= p, jnp.int32(0), jnp.int32(_INT_MIN))

    def body(i, t):
        cand = t | (jnp.int32(1) << (jnp.int32(30) - i))
        cnt = jnp.sum((k >= cand).astype(jnp.int32))
        return jnp.where(cnt >= p, cand, t)

    return jax.lax.fori_loop(0, 31, body, t0)


def _pool_kernel(x0ref, x1ref, plref, prref, oref):
    pl_m = plref[...]
    pr_m = prref[...]
    hw = oref.shape[2]
    cb = x0ref.shape[1]
    for s, xref in enumerate((x0ref, x1ref)):
        for i in range(cb):
            x = xref[0, i]
            t = jnp.dot(pl_m, x, preferred_element_type=jnp.float32)
            y = jnp.dot(t, pr_m, preferred_element_type=jnp.float32) * 0.25
            oref[0, s * cb + i] = y.astype(jnp.bfloat16).reshape(hw)


def _mlp_kernel(p, chunk, callref, cref, fref, w1fref, w1cref,
                w2ref, w3ref, b1ref, b2ref, b3ref, oref, ksref, tsref):
    b = pl.program_id(1)

    @pl.when(b == 0)
    def _():
        k_full = _sort_key(_uncertainty(callref[0]))
        ksref[...] = k_full
        tsref[0] = _find_threshold(k_full, p)

    c = cref[0]
    f = fref[0]
    k = ksref[:, pl.ds(b * chunk, chunk)]
    maskf = (k >= tsref[0]).astype(jnp.float32)
    h1 = jnp.maximum(
        jnp.dot(w1fref[...], f, preferred_element_type=jnp.float32)
        + jnp.dot(w1cref[...], c, preferred_element_type=jnp.float32)
        + b1ref[...], 0.0)
    h2 = jnp.maximum(
        jnp.dot(w2ref[...], h1.astype(jnp.bfloat16),
                preferred_element_type=jnp.float32)
        + b2ref[...], 0.0)
    lg = jnp.dot(w3ref[...], h2, preferred_element_type=jnp.float32) + b3ref[...]
    oref[0] = lg * maskf + c * (1.0 - maskf)


def kernel(coarse_logits, fine_features, w1, b1, w2, b2, w3, b3):
    N, C, H, W = coarse_logits.shape
    _, Cf, Hf, Wf = fine_features.shape
    HW = H * W
    P = min(HW, _NUM_POINTS)
    hidden = w1.shape[0]
    coarse3 = coarse_logits.reshape(N, C, HW)

    pl_mat = jnp.repeat(jnp.eye(Hf // 2, dtype=jnp.float32), 2, axis=1)
    pr_mat = jnp.repeat(jnp.eye(Wf // 2, dtype=jnp.float32), 2, axis=0)
    cb = 16
    step_c = 2 * cb
    fine_spec = lambda i: pl.BlockSpec(
        (1, cb, Hf, Wf), lambda n, j, i=i: (n, 2 * j + i, 0, 0))
    pooled = pl.pallas_call(
        _pool_kernel,
        grid=(N, Cf // step_c),
        in_specs=[fine_spec(0), fine_spec(1),
                  pl.BlockSpec((Hf // 2, Hf), lambda n, j: (0, 0)),
                  pl.BlockSpec((Wf, Wf // 2), lambda n, j: (0, 0))],
        out_specs=pl.BlockSpec((1, step_c, HW), lambda n, j: (n, j, 0)),
        out_shape=jax.ShapeDtypeStruct((N, Cf, HW), jnp.bfloat16),
    )(fine_features, fine_features, pl_mat, pr_mat)

    B = 4096
    nb = HW // B
    w1f = w1[:, :Cf].astype(jnp.bfloat16)
    w1c = w1[:, Cf:]
    w2b = w2.astype(jnp.bfloat16)
    refined = pl.pallas_call(
        functools.partial(_mlp_kernel, P, B),
        grid=(N, nb),
        in_specs=[
            pl.BlockSpec((1, C, HW), lambda n, b: (n, 0, 0)),
            pl.BlockSpec((1, C, B), lambda n, b: (n, 0, b)),
            pl.BlockSpec((1, Cf, B), lambda n, b: (n, 0, b)),
            pl.BlockSpec((hidden, Cf), lambda n, b: (0, 0)),
            pl.BlockSpec((hidden, C), lambda n, b: (0, 0)),
            pl.BlockSpec((hidden, hidden), lambda n, b: (0, 0)),
            pl.BlockSpec((C, hidden), lambda n, b: (0, 0)),
            pl.BlockSpec((hidden, 1), lambda n, b: (0, 0)),
            pl.BlockSpec((hidden, 1), lambda n, b: (0, 0)),
            pl.BlockSpec((C, 1), lambda n, b: (0, 0)),
        ],
        out_specs=pl.BlockSpec((1, C, B), lambda n, b: (n, 0, b)),
        out_shape=jax.ShapeDtypeStruct((N, C, HW), jnp.float32),
        scratch_shapes=[pltpu.VMEM((1, HW), jnp.int32),
                        pltpu.SMEM((1,), jnp.int32)],
    )(coarse3, coarse3, pooled, w1f, w1c, w2b, w3,
      b1.reshape(hidden, 1), b2.reshape(hidden, 1), b3.reshape(C, 1))
    return refined.reshape(N, C, H, W)


# speedup vs baseline: 1.0901x; 1.0056x over previous
import functools

import jax
import jax.numpy as jnp
from jax.experimental import pallas as pl
from jax.experimental.pallas import tpu as pltpu

_NUM_POINTS = 8192
_INT_MIN = -2147483648


def _uncertainty(x):
    m1 = jnp.max(x, axis=0, keepdims=True)
    eq = x == m1
    cnt = jnp.sum(eq.astype(jnp.int32), axis=0, keepdims=True)
    m2 = jnp.max(jnp.where(eq, -jnp.inf, x), axis=0, keepdims=True)
    second = jnp.where(cnt > 1, m1, m2)
    return second - m1


def _sort_key(u):
    b = jax.lax.bitcast_convert_type(u, jnp.int32)
    return jnp.where(b >= 0, b, ~(b & jnp.int32(0x7FFFFFFF)))


def _find_threshold(k, p):
    n_nonneg = jnp.sum((k >= 0).astype(jnp.int32))
    t0 = jnp.where(n_nonneg >= p, jnp.int32(0), jnp.int32(_INT_MIN))

    def body(i, t):
        cand = t | (jnp.int32(1) << (jnp.int32(30) - i))
        cnt = jnp.sum((k >= cand).astype(jnp.int32))
        return jnp.where(cnt >= p, cand, t)

    return jax.lax.fori_loop(0, 31, body, t0)


def _pool_kernel(x0ref, x1ref, plref, prref, oref):
    pl_m = plref[...]
    pr_m = prref[...]
    hw = oref.shape[2]
    cb = x0ref.shape[1]
    for s, xref in enumerate((x0ref, x1ref)):
        for i in range(cb):
            x = xref[0, i]
            t = jnp.dot(pl_m, x, preferred_element_type=jnp.float32)
            y = jnp.dot(t, pr_m, preferred_element_type=jnp.float32) * 0.25
            oref[0, s * cb + i] = y.astype(jnp.bfloat16).reshape(hw)


def _mlp_kernel(p, chunk, callref, cref, fref, w1fref, w1cref,
                w2ref, w3ref, b1ref, b2ref, b3ref, oref, ksref, tsref):
    b = pl.program_id(1)

    @pl.when(b == 0)
    def _():
        k_full = _sort_key(_uncertainty(callref[0]))
        ksref[...] = k_full
        tsref[0] = _find_threshold(k_full, p)

    c = cref[0]
    f = fref[0]
    k = ksref[:, pl.ds(b * chunk, chunk)]
    mask = k >= tsref[0]
    h1 = jnp.maximum(
        jnp.dot(w1fref[...], f, preferred_element_type=jnp.float32)
        + jnp.dot(w1cref[...], c, preferred_element_type=jnp.float32)
        + b1ref[...], 0.0)
    h2 = jnp.maximum(
        jnp.dot(w2ref[...], h1.astype(jnp.bfloat16),
                preferred_element_type=jnp.float32)
        + b2ref[...], 0.0)
    lg = jnp.dot(w3ref[...], h2, preferred_element_type=jnp.float32) + b3ref[...]
    oref[0] = jnp.where(mask, lg, c)


def kernel(coarse_logits, fine_features, w1, b1, w2, b2, w3, b3):
    N, C, H, W = coarse_logits.shape
    _, Cf, Hf, Wf = fine_features.shape
    HW = H * W
    P = min(HW, _NUM_POINTS)
    hidden = w1.shape[0]
    coarse3 = coarse_logits.reshape(N, C, HW)

    pl_mat = jnp.repeat(jnp.eye(Hf // 2, dtype=jnp.float32), 2, axis=1)
    pr_mat = jnp.repeat(jnp.eye(Wf // 2, dtype=jnp.float32), 2, axis=0)
    cb = 16
    step_c = 2 * cb
    fine_spec = lambda i: pl.BlockSpec(
        (1, cb, Hf, Wf), lambda n, j, i=i: (n, 2 * j + i, 0, 0))
    pooled = pl.pallas_call(
        _pool_kernel,
        grid=(N, Cf // step_c),
        in_specs=[fine_spec(0), fine_spec(1),
                  pl.BlockSpec((Hf // 2, Hf), lambda n, j: (0, 0)),
                  pl.BlockSpec((Wf, Wf // 2), lambda n, j: (0, 0))],
        out_specs=pl.BlockSpec((1, step_c, HW), lambda n, j: (n, j, 0)),
        out_shape=jax.ShapeDtypeStruct((N, Cf, HW), jnp.bfloat16),
    )(fine_features, fine_features, pl_mat, pr_mat)

    B = 8192
    nb = HW // B
    w1f = w1[:, :Cf].astype(jnp.bfloat16)
    w1c = w1[:, Cf:]
    w2b = w2.astype(jnp.bfloat16)
    refined = pl.pallas_call(
        functools.partial(_mlp_kernel, P, B),
        grid=(N, nb),
        in_specs=[
            pl.BlockSpec((1, C, HW), lambda n, b: (n, 0, 0)),
            pl.BlockSpec((1, C, B), lambda n, b: (n, 0, b)),
            pl.BlockSpec((1, Cf, B), lambda n, b: (n, 0, b)),
            pl.BlockSpec((hidden, Cf), lambda n, b: (0, 0)),
            pl.BlockSpec((hidden, C), lambda n, b: (0, 0)),
            pl.BlockSpec((hidden, hidden), lambda n, b: (0, 0)),
            pl.BlockSpec((C, hidden), lambda n, b: (0, 0)),
            pl.BlockSpec((hidden, 1), lambda n, b: (0, 0)),
            pl.BlockSpec((hidden, 1), lambda n, b: (0, 0)),
            pl.BlockSpec((C, 1), lambda n, b: (0, 0)),
        ],
        out_specs=pl.BlockSpec((1, C, B), lambda n, b: (n, 0, b)),
        out_shape=jax.ShapeDtypeStruct((N, C, HW), jnp.float32),
        scratch_shapes=[pltpu.VMEM((1, HW), jnp.int32),
                        pltpu.SMEM((1,), jnp.int32)],
    )(coarse3, coarse3, pooled, w1f, w1c, w2b, w3,
      b1.reshape(hidden, 1), b2.reshape(hidden, 1), b3.reshape(C, 1))
    return refined.reshape(N, C, H, W)
